# baseline (speedup 1.0000x reference)
import functools

import jax
import jax.numpy as jnp
from jax import lax
from jax.experimental import pallas as pl
from jax.experimental.pallas import tpu as pltpu
from jax.experimental.pallas import tpu_sc as plsc

NUM_USER = 2000
NUM_ITEM = 8000
N = NUM_USER + NUM_ITEM
E_DIR = 2 * 320000
NC, NS, L = 2, 16, 16
CHUNK = 128
N_PAD = 10112
ZROWS = N_PAD // NS
ROWS_PT = 624

IB = 40
CH_A = 64
KA = IB * (-(-E_DIR // (NS * CH_A * IB)))
PAD_A = NS * KA * CH_A - E_DIR
CH_B = 64
KB = IB * (-(-E_DIR // (NC * NS * CH_B * IB)))
PAD_B = NC * NS * KB * CH_B - E_DIR

_lr = functools.partial(jax.nn.leaky_relu, negative_slope=0.01)


def _tc_a_body(pref, vfeat, wmlp, bmlp, wconv1, wlin1, blin1, idemb,
               xw_st, xhat):
    i = pl.program_id(0)
    tf = jnp.dot(vfeat[...], wmlp[...].T, preferred_element_type=jnp.float32)
    tf = tf + bmlp[...]
    x = jnp.where(i < 2, pref[...], tf)
    nrm = jnp.sqrt(jnp.sum(x * x, axis=1, keepdims=True))
    x = x / jnp.maximum(nrm, 1e-12)
    xw = jnp.dot(x, wconv1[...], preferred_element_type=jnp.float32)
    xw_st[...] = jnp.stack([xw[:, :128], xw[:, 128:]], axis=0)
    xh = _lr(jnp.dot(x, wlin1[...].T, preferred_element_type=jnp.float32)
             + blin1[...])
    xhat[...] = xh + idemb[...]


def _tc_b_body(h_st, xhat_in, wg, bg, wconv, wlin, blin, idemb,
               xw_out, xhat_out):
    h = jnp.concatenate([h_st[0], h_st[1]], axis=1)
    h = _lr(h)
    _tc_mid_tail(h, xhat_in, wg, bg, wconv, wlin, blin, idemb,
                 xw_out, xhat_out)


def _tc_c_body(h_in, xhat_in, wg, bg, wconv, wlin, blin, idemb,
               xw_out, xhat_out):
    h = _lr(h_in[0] + h_in[1])
    _tc_mid_tail(h, xhat_in, wg, bg, wconv, wlin, blin, idemb,
                 xw_out, xhat_out)


def _tc_mid_tail(h, xhat_in, wg, bg, wconv, wlin, blin, idemb,
                 xw_out, xhat_out):
    g = jnp.dot(h, wg[...].T, preferred_element_type=jnp.float32) + bg[...]
    x = _lr(g + xhat_in[...])
    xw_out[...] = jnp.dot(x, wconv[...], preferred_element_type=jnp.float32)
    xh = _lr(jnp.dot(x, wlin[...].T, preferred_element_type=jnp.float32)
             + blin[...])
    xhat_out[...] = xh + idemb[...]


def _tc_d_body(h_in, xhat_in, wg, bg, out):
    h = _lr(h_in[0] + h_in[1])
    g = jnp.dot(h, wg[...].T, preferred_element_type=jnp.float32) + bg[...]
    out[...] = _lr(g + xhat_in[...])


_BLK = 1000
_GRID = N // _BLK


def _full(shape):
    return pl.BlockSpec(shape, lambda i: tuple(0 for _ in shape))


def _rows(width):
    return pl.BlockSpec((_BLK, width), lambda i: (i, 0))


def _stacked(width):
    return pl.BlockSpec((2, _BLK, width), lambda i: (0, i, 0))


def _sc_conv_body(n_chunks, nbuf, ch, xw, src, dst, zeros, out,
                  idx_src, idx_dst, rows, accum, *sems):
    semg, sems_ = sems[:nbuf], sems[nbuf:]
    c = lax.axis_index("c")
    s = lax.axis_index("s")
    w = c * NS + s
    pltpu.sync_copy(zeros, accum.at[pl.ds(s * ZROWS, ZROWS)])
    plsc.subcore_barrier()

    def wait_g(b):
        pltpu.make_async_copy(xw.at[idx_src.at[0]], rows.at[b],
                              semg[b]).wait()

    def wait_s(b):
        pltpu.make_async_copy(rows.at[b], accum.at[idx_dst.at[0]],
                              sems_[b]).wait()

    def block(blk, carry):
        pltpu.sync_copy(src.at[w, pl.ds(blk * IB, IB)], idx_src)
        pltpu.sync_copy(dst.at[w, pl.ds(blk * IB, IB)], idx_dst)
        for b in range(nbuf):
            pltpu.async_copy(xw.at[idx_src.at[b]], rows.at[b], semg[b])
        for b in range(nbuf):
            wait_g(b)
            pltpu.async_copy(rows.at[b], accum.at[idx_dst.at[b]], sems_[b],
                             add=True)

        def body(t, carry2):
            j = t * nbuf
            for b in range(nbuf):
                wait_s(b)
                pltpu.async_copy(xw.at[idx_src.at[j + b]], rows.at[b],
                                 semg[b])
            for b in range(nbuf):
                wait_g(b)
                pltpu.async_copy(rows.at[b], accum.at[idx_dst.at[j + b]],
                                 sems_[b], add=True)
            return carry2

        lax.fori_loop(1, IB // nbuf, body, 0)
        for b in range(nbuf):
            wait_s(b)
        return carry

    lax.fori_loop(0, n_chunks // IB, block, 0)
    plsc.subcore_barrier()

    @pl.when(s < NS - 1)
    def _():
        pltpu.sync_copy(accum.at[pl.ds(s * ROWS_PT, ROWS_PT)],
                        out.at[pl.ds(c * N + s * ROWS_PT, ROWS_PT)])

    @pl.when(s == NS - 1)
    def _():
        last = (NS - 1) * ROWS_PT
        pltpu.sync_copy(accum.at[pl.ds(last, N - last)],
                        out.at[pl.ds(c * N + last, N - last)])


def _make_sc_conv(n_chunks, d, nbuf, ch=CHUNK):
    mesh = plsc.VectorSubcoreMesh(core_axis_name="c", subcore_axis_name="s")
    return pl.kernel(
        functools.partial(_sc_conv_body, n_chunks, nbuf, ch),
        out_type=jax.ShapeDtypeStruct((2 * N, d), jnp.float32),
        mesh=mesh,
        compiler_params=pltpu.CompilerParams(use_tc_tiling_on_sc=False),
        scratch_types=[
            pltpu.VMEM((IB, ch), jnp.int32),
            pltpu.VMEM((IB, ch), jnp.int32),
            pltpu.VMEM((nbuf, ch, d), jnp.float32),
            pltpu.VMEM_SHARED((N_PAD, d), jnp.float32),
        ] + [pltpu.SemaphoreType.DMA] * (2 * nbuf),
    )


def kernel(v_feat, edge_index, preference, W_mlp, b_mlp,
           W_conv1, W_lin1, b_lin1, W_g1, b_g1,
           W_conv2, W_lin2, b_lin2, W_g2, b_g2,
           W_conv3, W_lin3, b_lin3, W_g3, b_g3,
           id_embedding):
    f32 = jnp.float32

    e0 = edge_index[:, 0]
    e1 = edge_index[:, 1]
    src_dir = jnp.concatenate([e0, e1])
    dst_dir = jnp.concatenate([e1, e0])

    def _tiles(vals, pad_vals, nt, ch=CHUNK):
        t = jnp.concatenate([vals, pad_vals])
        return t.reshape(nt, -1, ch)

    pad_src_a = (jnp.arange(PAD_A, dtype=jnp.int32) * 7) % N
    pad_dst_a = N + (jnp.arange(PAD_A, dtype=jnp.int32) % (N_PAD - N))
    pad_src_b = (jnp.arange(PAD_B, dtype=jnp.int32) * 7) % N
    pad_dst_b = N + (jnp.arange(PAD_B, dtype=jnp.int32) % (N_PAD - N))

    src_a_t = _tiles(src_dir, pad_src_a, NS, CH_A)
    dst_a_t = _tiles(dst_dir, pad_dst_a, NS, CH_A)
    src_a = jnp.concatenate([src_a_t, src_a_t + N], axis=0)
    dst_a = jnp.concatenate([dst_a_t, dst_a_t], axis=0)

    src_b = _tiles(src_dir, pad_src_b, NC * NS, CH_B)
    dst_b = _tiles(dst_dir, pad_dst_b, NC * NS, CH_B)

    zeros_a = jnp.zeros((ZROWS, 128), f32)
    zeros_b = jnp.zeros((ZROWS, 64), f32)

    b_mlp2 = b_mlp.reshape(1, 256)
    b_lin1_2 = b_lin1.reshape(1, 64)
    b_g1_2 = b_g1.reshape(1, 64)
    b_lin2_2 = b_lin2.reshape(1, 64)
    b_g2_2 = b_g2.reshape(1, 64)
    b_lin3_2 = b_lin3.reshape(1, 64)
    b_g3_2 = b_g3.reshape(1, 64)

    xw_st, xhat1 = pl.pallas_call(
        _tc_a_body,
        grid=(_GRID,),
        in_specs=[
            pl.BlockSpec((_BLK, 256), lambda i: (jnp.minimum(i, 1), 0)),
            pl.BlockSpec((_BLK, 128), lambda i: (jnp.maximum(i - 2, 0), 0)),
            _full((256, 128)), _full((1, 256)), _full((256, 256)),
            _full((64, 256)), _full((1, 64)), _rows(64),
        ],
        out_specs=[_stacked(128), _rows(64)],
        out_shape=[jax.ShapeDtypeStruct((2, N, 128), f32),
                   jax.ShapeDtypeStruct((N, 64), f32)],
    )(preference, v_feat, W_mlp, b_mlp2, W_conv1, W_lin1, b_lin1_2,
      id_embedding)

    xw2h = xw_st.reshape(2 * N, 128)
    h1_st = _make_sc_conv(KA, 128, 5, CH_A)(xw2h, src_a, dst_a,
                                   zeros_a).reshape(2, N, 128)

    xw2, xhat2 = pl.pallas_call(
        _tc_b_body,
        grid=(_GRID,),
        in_specs=[
            _stacked(128), _rows(64),
            _full((64, 256)), _full((1, 64)), _full((64, 64)),
            _full((64, 64)), _full((1, 64)), _rows(64),
        ],
        out_specs=[_rows(64), _rows(64)],
        out_shape=[jax.ShapeDtypeStruct((N, 64), f32),
                   jax.ShapeDtypeStruct((N, 64), f32)],
    )(h1_st, xhat1, W_g1, b_g1_2, W_conv2, W_lin2, b_lin2_2, id_embedding)

    h2_st = _make_sc_conv(KB, 64, 8, CH_B)(xw2, src_b, dst_b,
                                  zeros_b).reshape(2, N, 64)

    xw3, xhat3 = pl.pallas_call(
        _tc_c_body,
        grid=(_GRID,),
        in_specs=[
            _stacked(64), _rows(64),
            _full((64, 64)), _full((1, 64)), _full((64, 64)),
            _full((64, 64)), _full((1, 64)), _rows(64),
        ],
        out_specs=[_rows(64), _rows(64)],
        out_shape=[jax.ShapeDtypeStruct((N, 64), f32),
                   jax.ShapeDtypeStruct((N, 64), f32)],
    )(h2_st, xhat2, W_g2, b_g2_2, W_conv3, W_lin3, b_lin3_2, id_embedding)

    h3_st = _make_sc_conv(KB, 64, 8, CH_B)(xw3, src_b, dst_b,
                                  zeros_b).reshape(2, N, 64)

    out = pl.pallas_call(
        _tc_d_body,
        grid=(_GRID,),
        in_specs=[_stacked(64), _rows(64), _full((64, 64)), _full((1, 64))],
        out_specs=_rows(64),
        out_shape=jax.ShapeDtypeStruct((N, 64), f32),
    )(h3_st, xhat3, W_g3, b_g3_2)
    return out


# speedup vs baseline: 1.0606x; 1.0606x over previous
import functools

import jax
import jax.numpy as jnp
from jax import lax
from jax.experimental import pallas as pl
from jax.experimental.pallas import tpu as pltpu
from jax.experimental.pallas import tpu_sc as plsc

NUM_USER = 2000
NUM_ITEM = 8000
N = NUM_USER + NUM_ITEM
E_DIR = 2 * 320000
NC, NS, L = 2, 16, 16
CHUNK = 128
N_PAD = 10112
ZROWS = N_PAD // NS
ROWS_PT = 624

IB = 40
CH_A = 64
KA = IB * (-(-E_DIR // (NS * CH_A * IB)))
PAD_A = NS * KA * CH_A - E_DIR
CH_B = 64
KB = IB * (-(-E_DIR // (NC * NS * CH_B * IB)))
PAD_B = NC * NS * KB * CH_B - E_DIR

_lr = functools.partial(jax.nn.leaky_relu, negative_slope=0.01)


def _tc_a_body(pref, vfeat, wmlp, bmlp, wconv1, wlin1, blin1, idemb,
               xw_a, xw_b, xhat):
    i = pl.program_id(0)
    tf = jnp.dot(vfeat[...], wmlp[...].T, preferred_element_type=jnp.float32)
    tf = tf + bmlp[...]
    x = jnp.where(i < 2, pref[...], tf)
    nrm = jnp.sqrt(jnp.sum(x * x, axis=1, keepdims=True))
    x = x / jnp.maximum(nrm, 1e-12)
    xw = jnp.dot(x, wconv1[...], preferred_element_type=jnp.float32)
    xw_a[...] = xw[:, :128]
    xw_b[...] = xw[:, 128:]
    xh = _lr(jnp.dot(x, wlin1[...].T, preferred_element_type=jnp.float32)
             + blin1[...])
    xhat[...] = xh + idemb[...]


def _tc_b_body(h_st, xhat_in, wg, bg, wconv, wlin, blin, idemb,
               xw_out, xhat_out):
    h = jnp.concatenate([h_st[0], h_st[1]], axis=1)
    h = _lr(h)
    _tc_mid_tail(h, xhat_in, wg, bg, wconv, wlin, blin, idemb,
                 xw_out, xhat_out)


def _tc_c_body(h_in, xhat_in, wg, bg, wconv, wlin, blin, idemb,
               xw_out, xhat_out):
    h = _lr(h_in[0] + h_in[1])
    _tc_mid_tail(h, xhat_in, wg, bg, wconv, wlin, blin, idemb,
                 xw_out, xhat_out)


def _tc_mid_tail(h, xhat_in, wg, bg, wconv, wlin, blin, idemb,
                 xw_out, xhat_out):
    g = jnp.dot(h, wg[...].T, preferred_element_type=jnp.float32) + bg[...]
    x = _lr(g + xhat_in[...])
    xw_out[...] = jnp.dot(x, wconv[...], preferred_element_type=jnp.float32)
    xh = _lr(jnp.dot(x, wlin[...].T, preferred_element_type=jnp.float32)
             + blin[...])
    xhat_out[...] = xh + idemb[...]


def _tc_d_body(h_in, xhat_in, wg, bg, out):
    h = _lr(h_in[0] + h_in[1])
    g = jnp.dot(h, wg[...].T, preferred_element_type=jnp.float32) + bg[...]
    out[...] = _lr(g + xhat_in[...])


_BLK = 1000
_GRID = N // _BLK


def _full(shape):
    return pl.BlockSpec(shape, lambda i: tuple(0 for _ in shape))


def _rows(width):
    return pl.BlockSpec((_BLK, width), lambda i: (i, 0))


def _stacked(width):
    return pl.BlockSpec((2, _BLK, width), lambda i: (0, i, 0))


def _conv_pipeline(n_chunks, nbuf, xw, src, dst, plane,
                   idx_src, idx_dst, rows, accum, semg, sems_):
    def wait_g(b):
        pltpu.make_async_copy(xw.at[idx_src.at[0]], rows.at[b],
                              semg[b]).wait()

    def wait_s(b):
        pltpu.make_async_copy(rows.at[b], accum.at[idx_dst.at[0]],
                              sems_[b]).wait()

    def block(blk, carry):
        pltpu.sync_copy(src.at[plane, pl.ds(blk * IB, IB)], idx_src)
        pltpu.sync_copy(dst.at[plane, pl.ds(blk * IB, IB)], idx_dst)
        for b in range(nbuf):
            pltpu.async_copy(xw.at[idx_src.at[b]], rows.at[b], semg[b])
        for b in range(nbuf):
            wait_g(b)
            pltpu.async_copy(rows.at[b], accum.at[idx_dst.at[b]], sems_[b],
                             add=True)

        def body(t, carry2):
            j = t * nbuf
            for b in range(nbuf):
                wait_s(b)
                pltpu.async_copy(xw.at[idx_src.at[j + b]], rows.at[b],
                                 semg[b])
            for b in range(nbuf):
                wait_g(b)
                pltpu.async_copy(rows.at[b], accum.at[idx_dst.at[j + b]],
                                 sems_[b], add=True)
            return carry2

        lax.fori_loop(1, IB // nbuf, body, 0)
        for b in range(nbuf):
            wait_s(b)
        return carry

    lax.fori_loop(0, n_chunks // IB, block, 0)


def _copy_out(c, s, accum, out):
    @pl.when(s < NS - 1)
    def _():
        pltpu.sync_copy(accum.at[pl.ds(s * ROWS_PT, ROWS_PT)],
                        out.at[pl.ds(c * N + s * ROWS_PT, ROWS_PT)])

    @pl.when(s == NS - 1)
    def _():
        last = (NS - 1) * ROWS_PT
        pltpu.sync_copy(accum.at[pl.ds(last, N - last)],
                        out.at[pl.ds(c * N + last, N - last)])


def _sc_conv1_body(n_chunks, nbuf, xw_a, xw_b, src, dst, zeros, out,
                   idx_src, idx_dst, rows, accum, *sems):
    semg, sems_ = sems[:nbuf], sems[nbuf:]
    c = lax.axis_index("c")
    s = lax.axis_index("s")
    pltpu.sync_copy(zeros, accum.at[pl.ds(s * ZROWS, ZROWS)])
    plsc.subcore_barrier()

    @pl.when(c == 0)
    def _():
        _conv_pipeline(n_chunks, nbuf, xw_a, src, dst, s,
                       idx_src, idx_dst, rows, accum, semg, sems_)

    @pl.when(c == 1)
    def _():
        _conv_pipeline(n_chunks, nbuf, xw_b, src, dst, s,
                       idx_src, idx_dst, rows, accum, semg, sems_)

    plsc.subcore_barrier()
    _copy_out(c, s, accum, out)


def _sc_conv_body(n_chunks, nbuf, xw, src, dst, zeros, out,
                  idx_src, idx_dst, rows, accum, *sems):
    semg, sems_ = sems[:nbuf], sems[nbuf:]
    c = lax.axis_index("c")
    s = lax.axis_index("s")
    pltpu.sync_copy(zeros, accum.at[pl.ds(s * ZROWS, ZROWS)])
    plsc.subcore_barrier()
    _conv_pipeline(n_chunks, nbuf, xw, src, dst, c * NS + s,
                   idx_src, idx_dst, rows, accum, semg, sems_)
    plsc.subcore_barrier()
    _copy_out(c, s, accum, out)


def _sc_scratch(d, nbuf, ch):
    return [
        pltpu.VMEM((IB, ch), jnp.int32),
        pltpu.VMEM((IB, ch), jnp.int32),
        pltpu.VMEM((nbuf, ch, d), jnp.float32),
        pltpu.VMEM_SHARED((N_PAD, d), jnp.float32),
    ] + [pltpu.SemaphoreType.DMA] * (2 * nbuf)


_MESH = dict(core_axis_name="c", subcore_axis_name="s")


def _make_sc_conv1(n_chunks, nbuf, ch):
    return pl.kernel(
        functools.partial(_sc_conv1_body, n_chunks, nbuf),
        out_type=jax.ShapeDtypeStruct((2 * N, 128), jnp.float32),
        mesh=plsc.VectorSubcoreMesh(**_MESH),
        compiler_params=pltpu.CompilerParams(use_tc_tiling_on_sc=False),
        scratch_types=_sc_scratch(128, nbuf, ch),
    )


def _make_sc_conv(n_chunks, nbuf, ch):
    return pl.kernel(
        functools.partial(_sc_conv_body, n_chunks, nbuf),
        out_type=jax.ShapeDtypeStruct((2 * N, 64), jnp.float32),
        mesh=plsc.VectorSubcoreMesh(**_MESH),
        compiler_params=pltpu.CompilerParams(use_tc_tiling_on_sc=False),
        scratch_types=_sc_scratch(64, nbuf, ch),
    )


def kernel(v_feat, edge_index, preference, W_mlp, b_mlp,
           W_conv1, W_lin1, b_lin1, W_g1, b_g1,
           W_conv2, W_lin2, b_lin2, W_g2, b_g2,
           W_conv3, W_lin3, b_lin3, W_g3, b_g3,
           id_embedding):
    f32 = jnp.float32

    e0 = edge_index[:, 0]
    e1 = edge_index[:, 1]
    src_dir = jnp.concatenate([e0, e1])
    dst_dir = jnp.concatenate([e1, e0])

    pad_src = (jnp.arange(PAD_A, dtype=jnp.int32) * 7) % N
    pad_dst = N + (jnp.arange(PAD_A, dtype=jnp.int32) % (N_PAD - N))
    flat_src = jnp.concatenate([src_dir, pad_src])
    flat_dst = jnp.concatenate([dst_dir, pad_dst])
    src_a = flat_src.reshape(NS, KA, CH_A)
    dst_a = flat_dst.reshape(NS, KA, CH_A)
    src_b = flat_src.reshape(NC * NS, KB, CH_B)
    dst_b = flat_dst.reshape(NC * NS, KB, CH_B)

    zeros_a = jnp.zeros((ZROWS, 128), f32)
    zeros_b = jnp.zeros((ZROWS, 64), f32)

    b_mlp2 = b_mlp.reshape(1, 256)
    b_lin1_2 = b_lin1.reshape(1, 64)
    b_g1_2 = b_g1.reshape(1, 64)
    b_lin2_2 = b_lin2.reshape(1, 64)
    b_g2_2 = b_g2.reshape(1, 64)
    b_lin3_2 = b_lin3.reshape(1, 64)
    b_g3_2 = b_g3.reshape(1, 64)

    xw_a, xw_b, xhat1 = pl.pallas_call(
        _tc_a_body,
        grid=(_GRID,),
        in_specs=[
            pl.BlockSpec((_BLK, 256), lambda i: (jnp.minimum(i, 1), 0)),
            pl.BlockSpec((_BLK, 128), lambda i: (jnp.maximum(i - 2, 0), 0)),
            _full((256, 128)), _full((1, 256)), _full((256, 256)),
            _full((64, 256)), _full((1, 64)), _rows(64),
        ],
        out_specs=[_rows(128), _rows(128), _rows(64)],
        out_shape=[jax.ShapeDtypeStruct((N, 128), f32),
                   jax.ShapeDtypeStruct((N, 128), f32),
                   jax.ShapeDtypeStruct((N, 64), f32)],
    )(preference, v_feat, W_mlp, b_mlp2, W_conv1, W_lin1, b_lin1_2,
      id_embedding)

    h1_st = _make_sc_conv1(KA, 5, CH_A)(xw_a, xw_b, src_a, dst_a,
                                        zeros_a).reshape(2, N, 128)

    xw2, xhat2 = pl.pallas_call(
        _tc_b_body,
        grid=(_GRID,),
        in_specs=[
            _stacked(128), _rows(64),
            _full((64, 256)), _full((1, 64)), _full((64, 64)),
            _full((64, 64)), _full((1, 64)), _rows(64),
        ],
        out_specs=[_rows(64), _rows(64)],
        out_shape=[jax.ShapeDtypeStruct((N, 64), f32),
                   jax.ShapeDtypeStruct((N, 64), f32)],
    )(h1_st, xhat1, W_g1, b_g1_2, W_conv2, W_lin2, b_lin2_2, id_embedding)

    h2_st = _make_sc_conv(KB, 8, CH_B)(xw2, src_b, dst_b,
                                  zeros_b).reshape(2, N, 64)

    xw3, xhat3 = pl.pallas_call(
        _tc_c_body,
        grid=(_GRID,),
        in_specs=[
            _stacked(64), _rows(64),
            _full((64, 64)), _full((1, 64)), _full((64, 64)),
            _full((64, 64)), _full((1, 64)), _rows(64),
        ],
        out_specs=[_rows(64), _rows(64)],
        out_shape=[jax.ShapeDtypeStruct((N, 64), f32),
                   jax.ShapeDtypeStruct((N, 64), f32)],
    )(h2_st, xhat2, W_g2, b_g2_2, W_conv3, W_lin3, b_lin3_2, id_embedding)

    h3_st = _make_sc_conv(KB, 8, CH_B)(xw3, src_b, dst_b,
                                  zeros_b).reshape(2, N, 64)

    out = pl.pallas_call(
        _tc_d_body,
        grid=(_GRID,),
        in_specs=[_stacked(64), _rows(64), _full((64, 64)), _full((1, 64))],
        out_specs=_rows(64),
        out_shape=jax.ShapeDtypeStruct((N, 64), f32),
    )(h3_st, xhat3, W_g3, b_g3_2)
    return out
